# baseline (speedup 1.0000x reference)
import functools

import jax
import jax.numpy as jnp
from jax import lax
from jax.experimental import pallas as pl
from jax.experimental.pallas import tpu as pltpu
from jax.experimental.pallas import tpu_sc as plsc

_D = 64
_SCALE = 8.0
_SPLIT = 500224
_B = 16384
_S = 50
_GP = 64
_NJ = 8
_NBUF = 4


def _tc_pair_transpose(table_t):
    def body(a_ref, b_ref, o_ref):
        o_ref[:, 0:64] = jnp.transpose(a_ref[...])
        o_ref[:, 64:128] = jnp.transpose(b_ref[...])

    nblk = _SPLIT // 256
    return pl.pallas_call(
        body,
        grid=(nblk,),
        in_specs=[
            pl.BlockSpec((64, 256), lambda i: (0, i)),
            pl.BlockSpec((64, 256), lambda i: (0, jnp.minimum(nblk + i, 3906))),
        ],
        out_specs=pl.BlockSpec((256, 128), lambda i: (i, 0)),
        out_shape=jax.ShapeDtypeStruct((_SPLIT, 128), jnp.float32),
    )(table_t, table_t)


def _rsqrt16(s):
    xi = lax.bitcast_convert_type(s, jnp.int32)
    yi = jnp.int32(0x5F3759DF) - lax.shift_right_logical(xi, 1)
    y = lax.bitcast_convert_type(yi, jnp.float32)
    xh = s * 0.5
    for _ in range(2):
        y = y * (1.5 - xh * y * y)
    return y


def _sc_lookup_normalize(xt, table2v):
    mesh = plsc.VectorSubcoreMesh(core_axis_name="c", subcore_axis_name="s")
    info = plsc.get_sparse_core_info()
    nc = info.num_cores
    bw = _B // (info.num_cores * info.num_subcores)
    assert bw == _NJ * _GP

    @functools.partial(
        pl.kernel,
        mesh=mesh,
        out_type=jax.ShapeDtypeStruct((_S, 8, _B // 128, 8, 128), jnp.float32),
        compiler_params=pltpu.CompilerParams(
            use_tc_tiling_on_sc=False, needs_layout_passes=False
        ),
        scratch_types=[
            pltpu.VMEM((_S, bw), jnp.int32),
            pltpu.VMEM((_S, bw), jnp.int32),
            pltpu.VMEM((_NBUF, _GP, _D), jnp.float32),
            pltpu.VMEM((_NBUF, 8, 8, _GP), jnp.float32),
            pltpu.SemaphoreType.DMA((_NBUF,)),
            pltpu.SemaphoreType.DMA((_NBUF,)),
        ],
    )
    def body(xt_hbm, table_hbm, out_hbm, xt_v, pidx_v, buf_v, tbuf_v, gsem, ssem):
        wid = lax.axis_index("s") * nc + lax.axis_index("c")
        b0 = wid * bw
        pltpu.sync_copy(xt_hbm.at[:, pl.ds(b0, bw)], xt_v)

        iota = lax.iota(jnp.int32, 16)

        def sxform(s, c):
            def ixform(i, c2):
                v = xt_v[s, pl.ds(i * 16, 16)]
                v2 = v + v
                pidx_v[s, pl.ds(i * 16, 16)] = jnp.where(
                    v >= _SPLIT, v2 - (2 * _SPLIT - 1), v2
                )
                return c2

            lax.fori_loop(0, bw // 16, ixform, 0)
            return c

        lax.fori_loop(0, _S, sxform, 0)

        def gcopy(s, j):
            return pltpu.make_async_copy(
                table_hbm.at[pidx_v.at[s, pl.ds(j * _GP, _GP)]],
                buf_v.at[j % _NBUF],
                gsem.at[j % _NBUF],
            )

        def scopy(s, j):
            bt = wid * 4 + j // 2
            return pltpu.make_async_copy(
                tbuf_v.at[j % _NBUF],
                out_hbm.at[s, :, bt, :, pl.ds((j % 2) * _GP, _GP)],
                ssem.at[j % _NBUF],
            )

        def normalize(j):
            b = j % _NBUF

            def rows16(g, carry):
                rows = iota + g * 16
                cols = [jnp.full((16,), c, jnp.int32) for c in range(_D)]
                acc = jnp.zeros((16,), jnp.float32)
                for c in range(_D):
                    cv = plsc.load_gather(buf_v.at[b], [rows, cols[c]])
                    acc = acc + cv * cv
                scale = _rsqrt16(jnp.maximum(acc, 1e-24)) * _SCALE
                for c in range(_D):
                    cv = plsc.load_gather(buf_v.at[b], [rows, cols[c]])
                    tbuf_v[b, c // 8, c % 8, pl.ds(g * 16, 16)] = cv * scale
                return carry

            lax.fori_loop(0, _GP // 16, rows16, 0)

        gcopy(0, 0).start()
        gcopy(0, 1).start()

        def sstep(s, carry):
            for j in range(_NJ):
                if j < _NJ - 2:
                    if j >= 2:
                        scopy(s, j - 2).wait()
                    else:

                        @pl.when(s >= 1)
                        def _drain():
                            scopy(s - 1, j + _NJ - 2).wait()

                    gcopy(s, j + 2).start()
                else:

                    @pl.when(s < _S - 1)
                    def _prefetch():
                        scopy(s, j - 2).wait()
                        gcopy(s + 1, j + 2 - _NJ).start()

                gcopy(s, j).wait()
                normalize(j)
                scopy(s, j).start()
            return carry

        lax.fori_loop(0, _S, sstep, 0)
        for j in range(_NJ - 4, _NJ):
            scopy(_S - 1, j).wait()

    return body(xt, table2v)


def kernel(x, raw_embedding):
    xt = jnp.transpose(x).astype(jnp.int32)
    table2 = _tc_pair_transpose(jnp.transpose(raw_embedding))
    table2v = table2.reshape(2 * _SPLIT, _D)
    out5 = _sc_lookup_normalize(xt, table2v)
    return jnp.transpose(out5, (2, 4, 0, 1, 3)).reshape(_B, _S, _D)


# speedup vs baseline: 1.4815x; 1.4815x over previous
import functools

import jax
import jax.numpy as jnp
from jax import lax
from jax.experimental import pallas as pl
from jax.experimental.pallas import tpu as pltpu
from jax.experimental.pallas import tpu_sc as plsc

_D = 64
_SCALE = 8.0
_SPLIT = 500224
_B = 16384
_S = 50
_GP = 64
_NJ = 8
_NBUF = 4


def _pair_table(table):
    left = table[:_SPLIT]
    right = jnp.pad(table[_SPLIT:], ((0, 2 * _SPLIT - 1000000), (0, 0)))
    return jnp.concatenate([left, right], axis=1)


def _permute16(x, idx):
    dnums = lax.GatherDimensionNumbers(
        offset_dims=(), collapsed_slice_dims=(0,), start_index_map=(0,)
    )
    return lax.gather(
        x,
        idx[:, None],
        dimension_numbers=dnums,
        slice_sizes=(1,),
        mode=lax.GatherScatterMode.PROMISE_IN_BOUNDS,
    )


def _lane_sum16(x, iota):
    for k in (8, 4, 2, 1):
        x = x + _permute16(x, iota ^ k)
    return x


def _rsqrt16(s):
    xi = lax.bitcast_convert_type(s, jnp.int32)
    yi = jnp.int32(0x5F3759DF) - lax.shift_right_logical(xi, 1)
    y = lax.bitcast_convert_type(yi, jnp.float32)
    xh = s * 0.5
    for _ in range(2):
        y = y * (1.5 - xh * y * y)
    return y


def _sc_lookup_normalize(xt, table2v):
    mesh = plsc.VectorSubcoreMesh(core_axis_name="c", subcore_axis_name="s")
    info = plsc.get_sparse_core_info()
    nc = info.num_cores
    bw = _B // (info.num_cores * info.num_subcores)
    assert bw == _NJ * _GP

    @functools.partial(
        pl.kernel,
        mesh=mesh,
        out_type=jax.ShapeDtypeStruct((_S, 8, _B // 128, 8, 128), jnp.float32),
        compiler_params=pltpu.CompilerParams(
            use_tc_tiling_on_sc=False, needs_layout_passes=False
        ),
        scratch_types=[
            pltpu.VMEM((_S, bw), jnp.int32),
            pltpu.VMEM((_S, bw), jnp.int32),
            pltpu.VMEM((_NBUF, _GP, _D), jnp.float32),
            pltpu.VMEM((_NBUF, 8, 8, _GP), jnp.float32),
            pltpu.SemaphoreType.DMA((_NBUF,)),
            pltpu.SemaphoreType.DMA((_NBUF,)),
        ],
    )
    def body(xt_hbm, table_hbm, out_hbm, xt_v, pidx_v, buf_v, tbuf_v, gsem, ssem):
        wid = lax.axis_index("s") * nc + lax.axis_index("c")
        b0 = wid * bw
        pltpu.sync_copy(xt_hbm.at[:, pl.ds(b0, bw)], xt_v)

        iota = lax.iota(jnp.int32, 16)

        def sxform(s, c):
            def ixform(i, c2):
                v = xt_v[s, pl.ds(i * 16, 16)]
                v2 = v + v
                pidx_v[s, pl.ds(i * 16, 16)] = jnp.where(
                    v >= _SPLIT, v2 - (2 * _SPLIT - 1), v2
                )
                return c2

            lax.fori_loop(0, bw // 16, ixform, 0)
            return c

        lax.fori_loop(0, _S, sxform, 0)

        def gcopy(s, j):
            return pltpu.make_async_copy(
                table_hbm.at[pidx_v.at[s, pl.ds(j * _GP, _GP)]],
                buf_v.at[j % _NBUF],
                gsem.at[j % _NBUF],
            )

        def scopy(s, j):
            bt = wid * 4 + j // 2
            return pltpu.make_async_copy(
                tbuf_v.at[j % _NBUF],
                out_hbm.at[s, :, bt, :, pl.ds((j % 2) * _GP, _GP)],
                ssem.at[j % _NBUF],
            )

        cbv = [lax.shift_right_logical(iota + m * 16, 3) for m in range(4)]
        civ = [lax.bitwise_and(iota + m * 16, 7) for m in range(4)]

        def normalize(j):
            b = j % _NBUF

            def row(r, carry):
                e = [buf_v[b, r, pl.ds(m * 16, 16)] for m in range(4)]
                acc = e[0] * e[0] + e[1] * e[1] + e[2] * e[2] + e[3] * e[3]
                ssq = jnp.maximum(_lane_sum16(acc, iota), 1e-24)
                scale = _rsqrt16(ssq) * _SCALE
                rb = jnp.full((16,), r, jnp.int32)
                for m in range(4):
                    plsc.store_scatter(
                        tbuf_v.at[b], [cbv[m], civ[m], rb], e[m] * scale
                    )
                return carry

            lax.fori_loop(0, _GP, row, 0)

        gcopy(0, 0).start()
        gcopy(0, 1).start()

        def sstep(s, carry):
            for j in range(_NJ):
                if j < _NJ - 2:
                    if j >= 2:
                        scopy(s, j - 2).wait()
                    else:

                        @pl.when(s >= 1)
                        def _drain():
                            scopy(s - 1, j + _NJ - 2).wait()

                    gcopy(s, j + 2).start()
                else:

                    @pl.when(s < _S - 1)
                    def _prefetch():
                        scopy(s, j - 2).wait()
                        gcopy(s + 1, j + 2 - _NJ).start()

                gcopy(s, j).wait()
                normalize(j)
                scopy(s, j).start()
            return carry

        lax.fori_loop(0, _S, sstep, 0)
        for j in range(_NJ - 4, _NJ):
            scopy(_S - 1, j).wait()

    return body(xt, table2v)


def kernel(x, raw_embedding):
    xt = jnp.transpose(x).astype(jnp.int32)
    table2 = _pair_table(raw_embedding)
    table2v = table2.reshape(2 * _SPLIT, _D)
    out5 = _sc_lookup_normalize(xt, table2v)
    return jnp.transpose(out5, (2, 4, 0, 1, 3)).reshape(_B, _S, _D)


# speedup vs baseline: 2.1107x; 1.4247x over previous
import functools

import jax
import jax.numpy as jnp
from jax import lax
from jax.experimental import pallas as pl
from jax.experimental.pallas import tpu as pltpu
from jax.experimental.pallas import tpu_sc as plsc

_D = 64
_SCALE = 8.0
_SPLIT = 500224
_B = 16384
_S = 50
_GP = 64
_NJ = 8
_NBUF = 4


def _pair_table(table):
    left = table[:_SPLIT]
    right = jnp.pad(table[_SPLIT:], ((0, 2 * _SPLIT - 1000000), (0, 0)))
    return jnp.concatenate([left, right], axis=1)


def _permute16(x, idx):
    dnums = lax.GatherDimensionNumbers(
        offset_dims=(), collapsed_slice_dims=(0,), start_index_map=(0,)
    )
    return lax.gather(
        x,
        idx[:, None],
        dimension_numbers=dnums,
        slice_sizes=(1,),
        mode=lax.GatherScatterMode.PROMISE_IN_BOUNDS,
    )


def _lane_sum16(x, iota):
    for k in (8, 4, 2, 1):
        x = x + _permute16(x, iota ^ k)
    return x


def _rsqrt16(s):
    xi = lax.bitcast_convert_type(s, jnp.int32)
    yi = jnp.int32(0x5F3759DF) - lax.shift_right_logical(xi, 1)
    y = lax.bitcast_convert_type(yi, jnp.float32)
    xh = s * 0.5
    for _ in range(2):
        y = y * (1.5 - xh * y * y)
    return y


def _sc_lookup_normalize(xt, table2v):
    mesh = plsc.VectorSubcoreMesh(core_axis_name="c", subcore_axis_name="s")
    info = plsc.get_sparse_core_info()
    nc = info.num_cores
    bw = _B // (info.num_cores * info.num_subcores)
    assert bw == _NJ * _GP

    @functools.partial(
        pl.kernel,
        mesh=mesh,
        out_type=jax.ShapeDtypeStruct((_S, 8, _B // 128, 8, 128), jnp.float32),
        compiler_params=pltpu.CompilerParams(
            use_tc_tiling_on_sc=False, needs_layout_passes=False
        ),
        scratch_types=[
            pltpu.VMEM((_S, bw), jnp.int32),
            pltpu.VMEM((_S, bw), jnp.int32),
            pltpu.VMEM((_NBUF, _GP, _D), jnp.float32),
            pltpu.VMEM((_NBUF, 8, 8, _GP + 1), jnp.float32),
            pltpu.SemaphoreType.DMA((_NBUF,)),
            pltpu.SemaphoreType.DMA((_NBUF,)),
        ],
    )
    def body(xt_hbm, table_hbm, out_hbm, xt_v, pidx_v, buf_v, tbuf_v, gsem, ssem):
        wid = lax.axis_index("s") * nc + lax.axis_index("c")
        b0 = wid * bw
        pltpu.sync_copy(xt_hbm.at[:, pl.ds(b0, bw)], xt_v)

        iota = lax.iota(jnp.int32, 16)

        def sxform(s, c):
            def ixform(i, c2):
                v = xt_v[s, pl.ds(i * 16, 16)]
                v2 = v + v
                pidx_v[s, pl.ds(i * 16, 16)] = jnp.where(
                    v >= _SPLIT, v2 - (2 * _SPLIT - 1), v2
                )
                return c2

            lax.fori_loop(0, bw // 16, ixform, 0)
            return c

        lax.fori_loop(0, _S, sxform, 0)

        def gcopy(s, j):
            return pltpu.make_async_copy(
                table_hbm.at[pidx_v.at[s, pl.ds(j * _GP, _GP)]],
                buf_v.at[j % _NBUF],
                gsem.at[j % _NBUF],
            )

        def scopy(s, j):
            bt = wid * 4 + j // 2
            return pltpu.make_async_copy(
                tbuf_v.at[j % _NBUF, :, :, pl.ds(0, _GP)],
                out_hbm.at[s, :, bt, :, pl.ds((j % 2) * _GP, _GP)],
                ssem.at[j % _NBUF],
            )

        cbv = [lax.shift_right_logical(iota + m * 16, 3) for m in range(4)]
        civ = [lax.bitwise_and(iota + m * 16, 7) for m in range(4)]

        def normalize(j):
            b = j % _NBUF

            def row(r, carry):
                e = [buf_v[b, r, pl.ds(m * 16, 16)] for m in range(4)]
                acc = e[0] * e[0] + e[1] * e[1] + e[2] * e[2] + e[3] * e[3]
                ssq = jnp.maximum(_lane_sum16(acc, iota), 1e-24)
                scale = _rsqrt16(ssq) * _SCALE
                rb = jnp.full((16,), r, jnp.int32)
                for m in range(4):
                    plsc.store_scatter(
                        tbuf_v.at[b], [cbv[m], civ[m], rb], e[m] * scale
                    )
                return carry

            lax.fori_loop(0, _GP, row, 0)

        gcopy(0, 0).start()
        gcopy(0, 1).start()

        def sstep(s, carry):
            for j in range(_NJ):
                if j < _NJ - 2:
                    if j >= 2:
                        scopy(s, j - 2).wait()
                    else:

                        @pl.when(s >= 1)
                        def _drain():
                            scopy(s - 1, j + _NJ - 2).wait()

                    gcopy(s, j + 2).start()
                else:

                    @pl.when(s < _S - 1)
                    def _prefetch():
                        scopy(s, j - 2).wait()
                        gcopy(s + 1, j + 2 - _NJ).start()

                gcopy(s, j).wait()
                normalize(j)
                scopy(s, j).start()
            return carry

        lax.fori_loop(0, _S, sstep, 0)
        for j in range(_NJ - 4, _NJ):
            scopy(_S - 1, j).wait()

    return body(xt, table2v)


def kernel(x, raw_embedding):
    xt = jnp.transpose(x).astype(jnp.int32)
    table2 = _pair_table(raw_embedding)
    table2v = table2.reshape(2 * _SPLIT, _D)
    out5 = _sc_lookup_normalize(xt, table2v)
    return jnp.transpose(out5, (2, 4, 0, 1, 3)).reshape(_B, _S, _D)


# speedup vs baseline: 2.1273x; 1.0078x over previous
import functools

import jax
import jax.numpy as jnp
from jax import lax
from jax.experimental import pallas as pl
from jax.experimental.pallas import tpu as pltpu
from jax.experimental.pallas import tpu_sc as plsc

_D = 64
_SCALE = 8.0
_SPLIT = 500224
_B = 16384
_S = 50
_GP = 128
_NJ = 4
_NBUF = 4


def _pair_table(table):
    left = table[:_SPLIT]
    right = jnp.pad(table[_SPLIT:], ((0, 2 * _SPLIT - 1000000), (0, 0)))
    return jnp.concatenate([left, right], axis=1)


def _permute16(x, idx):
    dnums = lax.GatherDimensionNumbers(
        offset_dims=(), collapsed_slice_dims=(0,), start_index_map=(0,)
    )
    return lax.gather(
        x,
        idx[:, None],
        dimension_numbers=dnums,
        slice_sizes=(1,),
        mode=lax.GatherScatterMode.PROMISE_IN_BOUNDS,
    )


def _lane_sum16(x, iota):
    for k in (8, 4, 2, 1):
        x = x + _permute16(x, iota ^ k)
    return x


def _rsqrt16(s):
    xi = lax.bitcast_convert_type(s, jnp.int32)
    yi = jnp.int32(0x5F3759DF) - lax.shift_right_logical(xi, 1)
    y = lax.bitcast_convert_type(yi, jnp.float32)
    xh = s * 0.5
    for _ in range(2):
        y = y * (1.5 - xh * y * y)
    return y


def _sc_lookup_normalize(xt, table2v):
    mesh = plsc.VectorSubcoreMesh(core_axis_name="c", subcore_axis_name="s")
    info = plsc.get_sparse_core_info()
    nc = info.num_cores
    bw = _B // (info.num_cores * info.num_subcores)
    assert bw == _NJ * _GP

    @functools.partial(
        pl.kernel,
        mesh=mesh,
        out_type=jax.ShapeDtypeStruct((_S, 8, _B // 128, 8, 128), jnp.float32),
        compiler_params=pltpu.CompilerParams(
            use_tc_tiling_on_sc=False, needs_layout_passes=False
        ),
        scratch_types=[
            pltpu.VMEM((_S, bw), jnp.int32),
            pltpu.VMEM((_NBUF, _GP, _D), jnp.float32),
            pltpu.VMEM((_NBUF, 8, 8, _GP + 1), jnp.float32),
            pltpu.SemaphoreType.DMA((_NBUF,)),
            pltpu.SemaphoreType.DMA((_NBUF,)),
        ],
    )
    def body(xt_hbm, table_hbm, out_hbm, xt_v, buf_v, tbuf_v, gsem, ssem):
        wid = lax.axis_index("s") * nc + lax.axis_index("c")
        b0 = wid * bw
        pltpu.sync_copy(xt_hbm.at[:, pl.ds(b0, bw)], xt_v)

        iota = lax.iota(jnp.int32, 16)

        def sxform(s, c):
            def ixform(i, c2):
                v = xt_v[s, pl.ds(i * 16, 16)]
                v2 = v + v
                xt_v[s, pl.ds(i * 16, 16)] = jnp.where(
                    v >= _SPLIT, v2 - (2 * _SPLIT - 1), v2
                )
                return c2

            lax.fori_loop(0, bw // 16, ixform, 0)
            return c

        lax.fori_loop(0, _S, sxform, 0)

        def gcopy(s, j):
            return pltpu.make_async_copy(
                table_hbm.at[xt_v.at[s, pl.ds(j * _GP, _GP)]],
                buf_v.at[j % _NBUF],
                gsem.at[j % _NBUF],
            )

        def scopy(s, j):
            bt = wid * _NJ + j
            return pltpu.make_async_copy(
                tbuf_v.at[j % _NBUF, :, :, pl.ds(0, _GP)],
                out_hbm.at[s, :, bt, :, :],
                ssem.at[j % _NBUF],
            )

        cbv = [lax.shift_right_logical(iota + m * 16, 3) for m in range(4)]
        civ = [lax.bitwise_and(iota + m * 16, 7) for m in range(4)]

        def normalize(j):
            b = j % _NBUF

            def row(r, carry):
                e = [buf_v[b, r, pl.ds(m * 16, 16)] for m in range(4)]
                acc = e[0] * e[0] + e[1] * e[1] + e[2] * e[2] + e[3] * e[3]
                ssq = jnp.maximum(_lane_sum16(acc, iota), 1e-24)
                scale = _rsqrt16(ssq) * _SCALE
                rb = jnp.full((16,), r, jnp.int32)
                for m in range(4):
                    plsc.store_scatter(
                        tbuf_v.at[b], [cbv[m], civ[m], rb], e[m] * scale
                    )
                return carry

            lax.fori_loop(0, _GP, row, 0)

        gcopy(0, 0).start()
        gcopy(0, 1).start()

        def sstep(s, carry):
            for j in range(_NJ):
                if j < _NJ - 2:
                    if j >= 2:
                        scopy(s, j - 2).wait()
                    else:

                        @pl.when(s >= 1)
                        def _drain():
                            scopy(s - 1, j + _NJ - 2).wait()

                    gcopy(s, j + 2).start()
                else:

                    @pl.when(s < _S - 1)
                    def _prefetch():
                        scopy(s, j - 2).wait()
                        gcopy(s + 1, j + 2 - _NJ).start()

                gcopy(s, j).wait()
                normalize(j)
                scopy(s, j).start()
            return carry

        lax.fori_loop(0, _S, sstep, 0)
        for j in range(_NJ - 4, _NJ):
            scopy(_S - 1, j).wait()

    return body(xt, table2v)


def kernel(x, raw_embedding):
    xt = jnp.transpose(x).astype(jnp.int32)
    table2 = _pair_table(raw_embedding)
    table2v = table2.reshape(2 * _SPLIT, _D)
    out5 = _sc_lookup_normalize(xt, table2v)
    return jnp.transpose(out5, (2, 4, 0, 1, 3)).reshape(_B, _S, _D)


# speedup vs baseline: 2.2981x; 1.0803x over previous
import functools

import jax
import jax.numpy as jnp
from jax import lax
from jax.experimental import pallas as pl
from jax.experimental.pallas import tpu as pltpu
from jax.experimental.pallas import tpu_sc as plsc

_D = 64
_SCALE = 8.0
_SPLIT = 500224
_B = 16384
_S = 50
_GP = 128
_NJ = 4
_NBUF = 4


def _pair_table(table):
    left = table[:_SPLIT]
    right = jnp.pad(table[_SPLIT:], ((0, 2 * _SPLIT - 1000000), (0, 0)))
    return jnp.concatenate([left, right], axis=1)


def _permute16(x, idx):
    dnums = lax.GatherDimensionNumbers(
        offset_dims=(), collapsed_slice_dims=(0,), start_index_map=(0,)
    )
    return lax.gather(
        x,
        idx[:, None],
        dimension_numbers=dnums,
        slice_sizes=(1,),
        mode=lax.GatherScatterMode.PROMISE_IN_BOUNDS,
    )


def _lane_sum16(x, iota):
    for k in (8, 4, 2, 1):
        x = x + _permute16(x, iota ^ k)
    return x


def _rsqrt16(s):
    xi = lax.bitcast_convert_type(s, jnp.int32)
    yi = jnp.int32(0x5F3759DF) - lax.shift_right_logical(xi, 1)
    y = lax.bitcast_convert_type(yi, jnp.float32)
    xh = s * 0.5
    y = y * (1.5 - xh * y * y)
    return y


def _sc_lookup_normalize(xt, table2v):
    mesh = plsc.VectorSubcoreMesh(core_axis_name="c", subcore_axis_name="s")
    info = plsc.get_sparse_core_info()
    nc = info.num_cores
    bw = _B // (info.num_cores * info.num_subcores)
    assert bw == _NJ * _GP

    @functools.partial(
        pl.kernel,
        mesh=mesh,
        out_type=jax.ShapeDtypeStruct((_S, 8, _B // 128, 8, 128), jnp.float32),
        compiler_params=pltpu.CompilerParams(
            use_tc_tiling_on_sc=False, needs_layout_passes=False
        ),
        scratch_types=[
            pltpu.VMEM((_S, bw), jnp.int32),
            pltpu.VMEM((_NBUF, _GP, _D), jnp.float32),
            pltpu.VMEM((_NBUF, 8, 8, _GP + 1), jnp.float32),
            pltpu.SemaphoreType.DMA((_NBUF,)),
            pltpu.SemaphoreType.DMA((_NBUF,)),
        ],
    )
    def body(xt_hbm, table_hbm, out_hbm, xt_v, buf_v, tbuf_v, gsem, ssem):
        wid = lax.axis_index("s") * nc + lax.axis_index("c")
        b0 = wid * bw
        pltpu.sync_copy(xt_hbm.at[:, pl.ds(b0, bw)], xt_v)

        iota = lax.iota(jnp.int32, 16)

        def sxform(s, c):
            def ixform(i, c2):
                v = xt_v[s, pl.ds(i * 16, 16)]
                v2 = v + v
                xt_v[s, pl.ds(i * 16, 16)] = jnp.where(
                    v >= _SPLIT, v2 - (2 * _SPLIT - 1), v2
                )
                return c2

            lax.fori_loop(0, bw // 16, ixform, 0)
            return c

        lax.fori_loop(0, _S, sxform, 0)

        def gcopy(s, j):
            return pltpu.make_async_copy(
                table_hbm.at[xt_v.at[s, pl.ds(j * _GP, _GP)]],
                buf_v.at[j % _NBUF],
                gsem.at[j % _NBUF],
            )

        def scopy(s, j):
            bt = wid * _NJ + j
            return pltpu.make_async_copy(
                tbuf_v.at[j % _NBUF, :, :, pl.ds(0, _GP)],
                out_hbm.at[s, :, bt, :, :],
                ssem.at[j % _NBUF],
            )

        cbv = [lax.shift_right_logical(iota + m * 16, 3) for m in range(4)]
        civ = [lax.bitwise_and(iota + m * 16, 7) for m in range(4)]

        def normalize(j):
            b = j % _NBUF

            def rows4(r4, carry):
                for k in range(4):
                    r = r4 * 4 + k
                    e = [buf_v[b, r, pl.ds(m * 16, 16)] for m in range(4)]
                    acc = e[0] * e[0] + e[1] * e[1] + e[2] * e[2] + e[3] * e[3]
                    ssq = jnp.maximum(_lane_sum16(acc, iota), 1e-24)
                    scale = _rsqrt16(ssq) * _SCALE
                    rb = jnp.full((16,), r, jnp.int32)
                    for m in range(4):
                        plsc.store_scatter(
                            tbuf_v.at[b], [cbv[m], civ[m], rb], e[m] * scale
                        )
                return carry

            lax.fori_loop(0, _GP // 4, rows4, 0)

        gcopy(0, 0).start()
        gcopy(0, 1).start()

        def sstep(s, carry):
            for j in range(_NJ):
                if j < _NJ - 2:
                    if j >= 2:
                        scopy(s, j - 2).wait()
                    else:

                        @pl.when(s >= 1)
                        def _drain():
                            scopy(s - 1, j + _NJ - 2).wait()

                    gcopy(s, j + 2).start()
                else:

                    @pl.when(s < _S - 1)
                    def _prefetch():
                        scopy(s, j - 2).wait()
                        gcopy(s + 1, j + 2 - _NJ).start()

                gcopy(s, j).wait()
                normalize(j)
                scopy(s, j).start()
            return carry

        lax.fori_loop(0, _S, sstep, 0)
        for j in range(_NJ - 4, _NJ):
            scopy(_S - 1, j).wait()

    return body(xt, table2v)


def kernel(x, raw_embedding):
    xt = jnp.transpose(x).astype(jnp.int32)
    table2 = _pair_table(raw_embedding)
    table2v = table2.reshape(2 * _SPLIT, _D)
    out5 = _sc_lookup_normalize(xt, table2v)
    return jnp.transpose(out5, (2, 4, 0, 1, 3)).reshape(_B, _S, _D)


# speedup vs baseline: 2.6035x; 1.1329x over previous
import functools

import jax
import jax.numpy as jnp
from jax import lax
from jax.experimental import pallas as pl
from jax.experimental.pallas import tpu as pltpu
from jax.experimental.pallas import tpu_sc as plsc

_D = 64
_G = 128
_SCALE = 8.0
_SPLIT = 500224


def _pair_table(table):
    left = table[:_SPLIT]
    right = jnp.pad(table[_SPLIT:], ((0, 2 * _SPLIT - 1000000), (0, 0)))
    return jnp.concatenate([left, right], axis=1)


def _permute16(x, idx):
    dnums = lax.GatherDimensionNumbers(
        offset_dims=(), collapsed_slice_dims=(0,), start_index_map=(0,)
    )
    return lax.gather(
        x,
        idx[:, None],
        dimension_numbers=dnums,
        slice_sizes=(1,),
        mode=lax.GatherScatterMode.PROMISE_IN_BOUNDS,
    )


def _lane_sum16(x):
    i = lax.iota(jnp.int32, 16)
    for k in (8, 4, 2, 1):
        x = x + _permute16(x, i ^ k)
    return x


def _rsqrt16(s):
    xi = lax.bitcast_convert_type(s, jnp.int32)
    yi = jnp.int32(0x5F3759DF) - lax.shift_right_logical(xi, 1)
    y = lax.bitcast_convert_type(yi, jnp.float32)
    xh = s * 0.5
    for _ in range(2):
        y = y * (1.5 - xh * y * y)
    return y


@functools.partial(jax.jit, static_argnames=("n_rows", "per_w"))
def _lookup_normalize(idx_flat, table, *, n_rows, per_w):
    mesh = plsc.VectorSubcoreMesh(core_axis_name="c", subcore_axis_name="s")
    info = plsc.get_sparse_core_info()
    nc = info.num_cores
    n_groups = per_w // _G

    nbuf = 4
    assert n_groups % nbuf == 0 and n_groups >= 2 * nbuf

    @functools.partial(
        pl.kernel,
        mesh=mesh,
        out_type=jax.ShapeDtypeStruct((n_rows, _D), jnp.float32),
        compiler_params=pltpu.CompilerParams(use_tc_tiling_on_sc=False),
        scratch_types=[
            pltpu.VMEM((per_w,), jnp.int32),
            pltpu.VMEM((nbuf, _G, _D), jnp.float32),
            pltpu.SemaphoreType.DMA((nbuf,)),
            pltpu.SemaphoreType.DMA((nbuf,)),
        ],
    )
    def body(idx_hbm, table_hbm, out_hbm, idx_v, buf_v, gsem, ssem):
        wid = lax.axis_index("s") * nc + lax.axis_index("c")
        base = wid * per_w
        pltpu.sync_copy(idx_hbm.at[pl.ds(base, per_w)], idx_v)

        def ixform(i, c2):
            v = idx_v[pl.ds(i * 16, 16)]
            v2 = v + v
            idx_v[pl.ds(i * 16, 16)] = jnp.where(
                v >= _SPLIT, v2 - (2 * _SPLIT - 1), v2
            )
            return c2

        lax.fori_loop(0, per_w // 16, ixform, 0)

        def gcopy(g, b):
            return pltpu.make_async_copy(
                table_hbm.at[idx_v.at[pl.ds(g * _G, _G)]],
                buf_v.at[b],
                gsem.at[b],
            )

        def scopy(g, b):
            return pltpu.make_async_copy(
                buf_v.at[b],
                out_hbm.at[pl.ds(base + g * _G, _G)],
                ssem.at[b],
            )

        def normalize_group(bref):
            def rows4(r4, c):
                for k in range(4):
                    r = r4 * 4 + k
                    v0 = bref[r, pl.ds(0, 16)]
                    v1 = bref[r, pl.ds(16, 16)]
                    v2 = bref[r, pl.ds(32, 16)]
                    v3 = bref[r, pl.ds(48, 16)]
                    acc = v0 * v0 + v1 * v1 + v2 * v2 + v3 * v3
                    ssq = jnp.maximum(_lane_sum16(acc), 1e-24)
                    scale = _rsqrt16(ssq) * _SCALE
                    bref[r, pl.ds(0, 16)] = v0 * scale
                    bref[r, pl.ds(16, 16)] = v1 * scale
                    bref[r, pl.ds(32, 16)] = v2 * scale
                    bref[r, pl.ds(48, 16)] = v3 * scale
                return c

            lax.fori_loop(0, _G // 4, rows4, 0)

        gcopy(0, 0).start()
        gcopy(1, 1).start()

        def outer(q, carry):
            for b in range(nbuf):
                g = q * nbuf + b
                nb = (b + 2) % nbuf

                @pl.when(g + 2 < n_groups)
                def _prefetch():
                    @pl.when(g >= 2)
                    def _drain_store():
                        scopy(g - 2, nb).wait()

                    gcopy(g + 2, nb).start()

                gcopy(g, b).wait()
                normalize_group(buf_v.at[b])
                scopy(g, b).start()
            return carry

        lax.fori_loop(0, n_groups // nbuf, outer, 0)
        scopy(n_groups - 2, (n_groups - 2) % nbuf).wait()
        scopy(n_groups - 1, (n_groups - 1) % nbuf).wait()

    return body(idx_flat, table)


def kernel(x, raw_embedding):
    b, s = x.shape
    n_rows = b * s
    info = plsc.get_sparse_core_info()
    nw = info.num_cores * info.num_subcores
    per_w = n_rows // nw
    assert per_w * nw == n_rows and per_w % _G == 0
    idx_flat = x.reshape(-1).astype(jnp.int32)
    table2v = _pair_table(raw_embedding).reshape(2 * _SPLIT, _D)
    out = _lookup_normalize(idx_flat, table2v, n_rows=n_rows, per_w=per_w)
    return out.reshape(b, s, _D)
